# Initial kernel scaffold; baseline (speedup 1.0000x reference)
import jax
import jax.numpy as jnp
from jax.experimental import pallas as pl


def kernel(x, edge_index, y, edge_attr, W1, b1, W2, b2):
    raise NotImplementedError("write your pallas kernel here")


# speedup vs baseline: 13.3503x; 13.3503x over previous
import functools

import jax
import jax.numpy as jnp
from jax import lax
from jax.experimental import pallas as pl
from jax.experimental.pallas import tpu as pltpu
from jax.experimental.pallas import tpu_sc as plsc

N = 10000
NPAD = 10240
DIN, DH, DOUT = 128, 64, 128
NC, NS = 2, 16
NW = NC * NS
C = 128
E = 320000
K = -(-E // (NW * C))
EPAD = NW * K * C
RPT = NPAD // NS

_mesh = plsc.VectorSubcoreMesh(core_axis_name="c", subcore_axis_name="s")
_sc_params = pltpu.CompilerParams(use_tc_tiling_on_sc=False)


def _deg_pass(dst_r):
    @functools.partial(
        pl.kernel,
        out_type=jax.ShapeDtypeStruct((NC, NPAD), jnp.float32),
        mesh=_mesh,
        scratch_types=[
            pltpu.VMEM((K, C), jnp.int32),
            pltpu.VMEM((C,), jnp.float32),
            pltpu.VMEM((RPT,), jnp.float32),
            pltpu.VMEM_SHARED((NPAD,), jnp.float32),
        ],
        compiler_params=_sc_params,
    )
    def degk(dst_hbm, out_hbm, dst_v, ones_v, z_v, acc_sh):
        c = lax.axis_index("c")
        s = lax.axis_index("s")
        wid = c * NS + s
        pltpu.sync_copy(dst_hbm.at[wid], dst_v)
        one16 = jnp.ones((16,), jnp.float32)
        zero16 = jnp.zeros((16,), jnp.float32)
        for j in range(0, C, 16):
            ones_v[pl.ds(j, 16)] = one16

        def zr(i, carry):
            z_v[pl.ds(i * 16, 16)] = zero16
            return carry

        lax.fori_loop(0, RPT // 16, zr, 0)
        pltpu.sync_copy(z_v, acc_sh.at[pl.ds(s * RPT, RPT)])
        plsc.subcore_barrier()

        def body(kk, carry):
            pltpu.sync_copy(ones_v, acc_sh.at[dst_v.at[kk]], add=True)
            return carry

        lax.fori_loop(0, K, body, 0)
        plsc.subcore_barrier()
        pltpu.sync_copy(acc_sh.at[pl.ds(s * RPT, RPT)],
                        out_hbm.at[c, pl.ds(s * RPT, RPT)])

    return degk(dst_r)


def _edge_pass(feat, src_r, dst_r, d):
    @functools.partial(
        pl.kernel,
        out_type=jax.ShapeDtypeStruct((NC, NPAD, d), jnp.float32),
        mesh=_mesh,
        scratch_types=[
            pltpu.VMEM((K, C), jnp.int32),
            pltpu.VMEM((K, C), jnp.int32),
            pltpu.VMEM((2, C, d), jnp.float32),
            pltpu.VMEM_SHARED((NPAD, d), jnp.float32),
        ],
        compiler_params=_sc_params,
    )
    def edgek(feat_hbm, src_hbm, dst_hbm, out_hbm, src_v, dst_v, rows_v, acc_sh):
        c = lax.axis_index("c")
        s = lax.axis_index("s")
        wid = c * NS + s
        pltpu.sync_copy(src_hbm.at[wid], src_v)
        pltpu.sync_copy(dst_hbm.at[wid], dst_v)

        zero16 = jnp.zeros((16,), jnp.float32)

        def zrow(r, carry):
            for j in range(0, d, 16):
                rows_v[0, r, pl.ds(j, 16)] = zero16
            return carry

        lax.fori_loop(0, C, zrow, 0)
        for i in range(RPT // C):
            pltpu.sync_copy(rows_v.at[0],
                            acc_sh.at[pl.ds(s * RPT + i * C, C)])
        plsc.subcore_barrier()

        def body(kk, carry):
            pltpu.sync_copy(feat_hbm.at[src_v.at[kk]], rows_v.at[0])
            pltpu.sync_copy(rows_v.at[0], acc_sh.at[dst_v.at[kk]], add=True)
            return carry

        lax.fori_loop(0, K, body, 0)
        plsc.subcore_barrier()
        pltpu.sync_copy(acc_sh.at[pl.ds(s * RPT, RPT)],
                        out_hbm.at[c, pl.ds(s * RPT, RPT)])

    return edgek(feat, src_r, dst_r)


_BR = 640


def _tc_a(xp, W1, degp_t):
    def body(x_ref, w_ref, dp_ref, g_ref, di_ref):
        dp = dp_ref[...]
        deg = dp[:, 0:1] + dp[:, 1:2] + 1.0
        dinv = lax.rsqrt(deg)
        h0 = jnp.dot(x_ref[...], w_ref[...], preferred_element_type=jnp.float32)
        g_ref[...] = h0 * dinv
        di_ref[...] = dinv

    grid = (NPAD // _BR,)
    return pl.pallas_call(
        body,
        grid=grid,
        in_specs=[
            pl.BlockSpec((_BR, DIN), lambda i: (i, 0)),
            pl.BlockSpec((DIN, DH), lambda i: (0, 0)),
            pl.BlockSpec((_BR, NC), lambda i: (i, 0)),
        ],
        out_specs=[
            pl.BlockSpec((_BR, DH), lambda i: (i, 0)),
            pl.BlockSpec((_BR, 1), lambda i: (i, 0)),
        ],
        out_shape=[
            jax.ShapeDtypeStruct((NPAD, DH), jnp.float32),
            jax.ShapeDtypeStruct((NPAD, 1), jnp.float32),
        ],
    )(xp, W1, degp_t)


def _tc_b(p0, p1, ghat, dinv, b1r, W2):
    def body(p0_ref, p1_ref, g_ref, di_ref, b_ref, w_ref, oa_ref, ob_ref):
        di = di_ref[...]
        h = di * (p0_ref[...] + p1_ref[...] + g_ref[...]) + b_ref[...]
        t = jnp.dot(h, w_ref[...], preferred_element_type=jnp.float32)
        hh = di * t
        oa_ref[...] = hh[:, :DOUT // 2]
        ob_ref[...] = hh[:, DOUT // 2:]

    grid = (NPAD // _BR,)
    return pl.pallas_call(
        body,
        grid=grid,
        in_specs=[
            pl.BlockSpec((_BR, DH), lambda i: (i, 0)),
            pl.BlockSpec((_BR, DH), lambda i: (i, 0)),
            pl.BlockSpec((_BR, DH), lambda i: (i, 0)),
            pl.BlockSpec((_BR, 1), lambda i: (i, 0)),
            pl.BlockSpec((1, DH), lambda i: (0, 0)),
            pl.BlockSpec((DH, DOUT), lambda i: (0, 0)),
        ],
        out_specs=[
            pl.BlockSpec((_BR, DOUT // 2), lambda i: (i, 0)),
            pl.BlockSpec((_BR, DOUT // 2), lambda i: (i, 0)),
        ],
        out_shape=[
            jax.ShapeDtypeStruct((NPAD, DOUT // 2), jnp.float32),
            jax.ShapeDtypeStruct((NPAD, DOUT // 2), jnp.float32),
        ],
    )(p0, p1, ghat, dinv, b1r, W2)


def _tc_c(qa, qb, ha, hb, dinv, b2r):
    def body(qa0_ref, qa1_ref, qb0_ref, qb1_ref, ha_ref, hb_ref, di_ref, b_ref, o_ref):
        di = di_ref[...]
        b = b_ref[...]
        left = di * (qa0_ref[...] + qa1_ref[...] + ha_ref[...]) + b[:, :DOUT // 2]
        right = di * (qb0_ref[...] + qb1_ref[...] + hb_ref[...]) + b[:, DOUT // 2:]
        o_ref[...] = jnp.concatenate([left, right], axis=-1)

    grid = (NPAD // _BR,)
    half = pl.BlockSpec((_BR, DOUT // 2), lambda i: (i, 0))
    return pl.pallas_call(
        body,
        grid=grid,
        in_specs=[half, half, half, half, half, half,
                  pl.BlockSpec((_BR, 1), lambda i: (i, 0)),
                  pl.BlockSpec((1, DOUT), lambda i: (0, 0))],
        out_specs=pl.BlockSpec((_BR, DOUT), lambda i: (i, 0)),
        out_shape=jax.ShapeDtypeStruct((NPAD, DOUT), jnp.float32),
    )(qa[0], qa[1], qb[0], qb[1], ha, hb, dinv, b2r)


def kernel(x, edge_index, y, edge_attr, W1, b1, W2, b2):
    src = edge_index[0]
    dst = edge_index[1]
    pad_e = EPAD - E
    pad_dst = N + (jnp.arange(pad_e, dtype=jnp.int32) % (NPAD - N))
    srcp = jnp.concatenate([src, jnp.zeros((pad_e,), jnp.int32)]).reshape(NW, K, C)
    dstp = jnp.concatenate([dst, pad_dst]).reshape(NW, K, C)
    xp = jnp.zeros((NPAD, DIN), jnp.float32).at[:N].set(x)

    degp = _deg_pass(dstp)
    ghat, dinv = _tc_a(xp, W1, degp.T)
    p = _edge_pass(ghat, srcp, dstp, DH)
    ha, hb = _tc_b(p[0], p[1], ghat, dinv, b1.reshape(1, DH), W2)
    qa = _edge_pass(ha, srcp, dstp, DOUT // 2)
    qb = _edge_pass(hb, srcp, dstp, DOUT // 2)
    outp = _tc_c(qa, qb, ha, hb, dinv, b2.reshape(1, DOUT))
    return outp[:N]


# speedup vs baseline: 14.9362x; 1.1188x over previous
import functools

import jax
import jax.numpy as jnp
from jax import lax
from jax.experimental import pallas as pl
from jax.experimental.pallas import tpu as pltpu
from jax.experimental.pallas import tpu_sc as plsc

N = 10000
NPAD = 10240
DIN, DH, DOUT = 128, 64, 128
NC, NS = 2, 16
NW = NC * NS
C = 128
E = 320000
K = -(-E // (NW * C))
EPAD = NW * K * C
RPT = NPAD // NS

_mesh = plsc.VectorSubcoreMesh(core_axis_name="c", subcore_axis_name="s")
_sc_params = pltpu.CompilerParams(use_tc_tiling_on_sc=False)


def _deg_pass(dst_r):
    @functools.partial(
        pl.kernel,
        out_type=jax.ShapeDtypeStruct((NC, NPAD), jnp.float32),
        mesh=_mesh,
        scratch_types=[
            pltpu.VMEM((K, C), jnp.int32),
            pltpu.VMEM((C,), jnp.float32),
            pltpu.VMEM((RPT,), jnp.float32),
            pltpu.VMEM_SHARED((NPAD,), jnp.float32),
        ],
        compiler_params=_sc_params,
    )
    def degk(dst_hbm, out_hbm, dst_v, ones_v, z_v, acc_sh):
        c = lax.axis_index("c")
        s = lax.axis_index("s")
        wid = c * NS + s
        pltpu.sync_copy(dst_hbm.at[wid], dst_v)
        one16 = jnp.ones((16,), jnp.float32)
        zero16 = jnp.zeros((16,), jnp.float32)
        for j in range(0, C, 16):
            ones_v[pl.ds(j, 16)] = one16

        def zr(i, carry):
            z_v[pl.ds(i * 16, 16)] = zero16
            return carry

        lax.fori_loop(0, RPT // 16, zr, 0)
        pltpu.sync_copy(z_v, acc_sh.at[pl.ds(s * RPT, RPT)])
        plsc.subcore_barrier()

        def body(kk, carry):
            pltpu.sync_copy(ones_v, acc_sh.at[dst_v.at[kk]], add=True)
            return carry

        lax.fori_loop(0, K, body, 0)
        plsc.subcore_barrier()
        pltpu.sync_copy(acc_sh.at[pl.ds(s * RPT, RPT)],
                        out_hbm.at[c, pl.ds(s * RPT, RPT)])

    return degk(dst_r)


def _edge_pass(feat, src_r, dst_r, d):
    @functools.partial(
        pl.kernel,
        out_type=jax.ShapeDtypeStruct((NC, NPAD, d), jnp.float32),
        mesh=_mesh,
        scratch_types=[
            pltpu.VMEM((K, C), jnp.int32),
            pltpu.VMEM((K, C), jnp.int32),
            pltpu.VMEM((2, C, d), jnp.float32),
            pltpu.VMEM_SHARED((NPAD, d), jnp.float32),
            pltpu.SemaphoreType.DMA,
            pltpu.SemaphoreType.DMA,
        ],
        compiler_params=_sc_params,
    )
    def edgek(feat_hbm, src_hbm, dst_hbm, out_hbm, src_v, dst_v, rows_v, acc_sh,
              sem0, sem1):
        c = lax.axis_index("c")
        s = lax.axis_index("s")
        wid = c * NS + s
        pltpu.sync_copy(src_hbm.at[wid], src_v)
        pltpu.sync_copy(dst_hbm.at[wid], dst_v)

        zero16 = jnp.zeros((16,), jnp.float32)

        def zrow(r, carry):
            for j in range(0, d, 16):
                rows_v[0, r, pl.ds(j, 16)] = zero16
            return carry

        lax.fori_loop(0, C, zrow, 0)
        for i in range(RPT // C):
            pltpu.sync_copy(rows_v.at[0],
                            acc_sh.at[pl.ds(s * RPT + i * C, C)])
        plsc.subcore_barrier()

        pltpu.async_copy(feat_hbm.at[src_v.at[0]], rows_v.at[0], sem0)

        def body(g, carry):
            k0 = 2 * g
            pltpu.make_async_copy(feat_hbm.at[src_v.at[k0]], rows_v.at[0], sem0).wait()
            pltpu.async_copy(feat_hbm.at[src_v.at[k0 + 1]], rows_v.at[1], sem1)
            pltpu.sync_copy(rows_v.at[0], acc_sh.at[dst_v.at[k0]], add=True)
            pltpu.make_async_copy(feat_hbm.at[src_v.at[k0 + 1]], rows_v.at[1], sem1).wait()
            pltpu.async_copy(feat_hbm.at[src_v.at[k0 + 2]], rows_v.at[0], sem0)
            pltpu.sync_copy(rows_v.at[1], acc_sh.at[dst_v.at[k0 + 1]], add=True)
            return carry

        lax.fori_loop(0, (K - 1) // 2, body, 0)
        pltpu.make_async_copy(feat_hbm.at[src_v.at[K - 1]], rows_v.at[0], sem0).wait()
        pltpu.sync_copy(rows_v.at[0], acc_sh.at[dst_v.at[K - 1]], add=True)
        plsc.subcore_barrier()
        pltpu.sync_copy(acc_sh.at[pl.ds(s * RPT, RPT)],
                        out_hbm.at[c, pl.ds(s * RPT, RPT)])

    return edgek(feat, src_r, dst_r)


_BR = 640


def _tc_a(xp, W1, degp_t):
    def body(x_ref, w_ref, dp_ref, g_ref, di_ref):
        dp = dp_ref[...]
        deg = dp[:, 0:1] + dp[:, 1:2] + 1.0
        dinv = lax.rsqrt(deg)
        h0 = jnp.dot(x_ref[...], w_ref[...], preferred_element_type=jnp.float32)
        g_ref[...] = h0 * dinv
        di_ref[...] = dinv

    grid = (NPAD // _BR,)
    return pl.pallas_call(
        body,
        grid=grid,
        in_specs=[
            pl.BlockSpec((_BR, DIN), lambda i: (i, 0)),
            pl.BlockSpec((DIN, DH), lambda i: (0, 0)),
            pl.BlockSpec((_BR, NC), lambda i: (i, 0)),
        ],
        out_specs=[
            pl.BlockSpec((_BR, DH), lambda i: (i, 0)),
            pl.BlockSpec((_BR, 1), lambda i: (i, 0)),
        ],
        out_shape=[
            jax.ShapeDtypeStruct((NPAD, DH), jnp.float32),
            jax.ShapeDtypeStruct((NPAD, 1), jnp.float32),
        ],
    )(xp, W1, degp_t)


def _tc_b(p0, p1, ghat, dinv, b1r, W2):
    def body(p0_ref, p1_ref, g_ref, di_ref, b_ref, w_ref, oa_ref, ob_ref):
        di = di_ref[...]
        h = di * (p0_ref[...] + p1_ref[...] + g_ref[...]) + b_ref[...]
        t = jnp.dot(h, w_ref[...], preferred_element_type=jnp.float32)
        hh = di * t
        oa_ref[...] = hh[:, :DOUT // 2]
        ob_ref[...] = hh[:, DOUT // 2:]

    grid = (NPAD // _BR,)
    return pl.pallas_call(
        body,
        grid=grid,
        in_specs=[
            pl.BlockSpec((_BR, DH), lambda i: (i, 0)),
            pl.BlockSpec((_BR, DH), lambda i: (i, 0)),
            pl.BlockSpec((_BR, DH), lambda i: (i, 0)),
            pl.BlockSpec((_BR, 1), lambda i: (i, 0)),
            pl.BlockSpec((1, DH), lambda i: (0, 0)),
            pl.BlockSpec((DH, DOUT), lambda i: (0, 0)),
        ],
        out_specs=[
            pl.BlockSpec((_BR, DOUT // 2), lambda i: (i, 0)),
            pl.BlockSpec((_BR, DOUT // 2), lambda i: (i, 0)),
        ],
        out_shape=[
            jax.ShapeDtypeStruct((NPAD, DOUT // 2), jnp.float32),
            jax.ShapeDtypeStruct((NPAD, DOUT // 2), jnp.float32),
        ],
    )(p0, p1, ghat, dinv, b1r, W2)


def _tc_c(qa, qb, ha, hb, dinv, b2r):
    def body(qa0_ref, qa1_ref, qb0_ref, qb1_ref, ha_ref, hb_ref, di_ref, b_ref, o_ref):
        di = di_ref[...]
        b = b_ref[...]
        left = di * (qa0_ref[...] + qa1_ref[...] + ha_ref[...]) + b[:, :DOUT // 2]
        right = di * (qb0_ref[...] + qb1_ref[...] + hb_ref[...]) + b[:, DOUT // 2:]
        o_ref[...] = jnp.concatenate([left, right], axis=-1)

    grid = (NPAD // _BR,)
    half = pl.BlockSpec((_BR, DOUT // 2), lambda i: (i, 0))
    return pl.pallas_call(
        body,
        grid=grid,
        in_specs=[half, half, half, half, half, half,
                  pl.BlockSpec((_BR, 1), lambda i: (i, 0)),
                  pl.BlockSpec((1, DOUT), lambda i: (0, 0))],
        out_specs=pl.BlockSpec((_BR, DOUT), lambda i: (i, 0)),
        out_shape=jax.ShapeDtypeStruct((NPAD, DOUT), jnp.float32),
    )(qa[0], qa[1], qb[0], qb[1], ha, hb, dinv, b2r)


def kernel(x, edge_index, y, edge_attr, W1, b1, W2, b2):
    src = edge_index[0]
    dst = edge_index[1]
    pad_e = EPAD - E
    pad_dst = N + (jnp.arange(pad_e, dtype=jnp.int32) % (NPAD - N))
    srcp = jnp.concatenate([src, jnp.zeros((pad_e,), jnp.int32)]).reshape(NW, K, C)
    dstp = jnp.concatenate([dst, pad_dst]).reshape(NW, K, C)
    xp = jnp.zeros((NPAD, DIN), jnp.float32).at[:N].set(x)

    degp = _deg_pass(dstp)
    ghat, dinv = _tc_a(xp, W1, degp.T)
    p = _edge_pass(ghat, srcp, dstp, DH)
    ha, hb = _tc_b(p[0], p[1], ghat, dinv, b1.reshape(1, DH), W2)
    qa = _edge_pass(ha, srcp, dstp, DOUT // 2)
    qb = _edge_pass(hb, srcp, dstp, DOUT // 2)
    outp = _tc_c(qa, qb, ha, hb, dinv, b2.reshape(1, DOUT))
    return outp[:N]
